# Initial kernel scaffold; baseline (speedup 1.0000x reference)
import jax
import jax.numpy as jnp
from jax.experimental import pallas as pl


def kernel(x, edge_index, W1, W2):
    raise NotImplementedError("write your pallas kernel here")


# speedup vs baseline: 23.6130x; 23.6130x over previous
import functools

import jax
import jax.numpy as jnp
from jax import lax
from jax.experimental import pallas as pl
from jax.experimental.pallas import tpu as pltpu
from jax.experimental.pallas import tpu_sc as plsc

_NC = 2
_NS = 16
_NW = _NC * _NS
_CHUNK = 128


def _sc_mesh():
    return plsc.VectorSubcoreMesh(core_axis_name="c", subcore_axis_name="s")


_DEG_W = 8


def _make_deg_kernel(K, npad, rows_per_tile):
    @functools.partial(
        pl.kernel,
        out_type=jax.ShapeDtypeStruct((_NC, npad, _DEG_W), jnp.float32),
        mesh=_sc_mesh(),
        scratch_types=[
            pltpu.VMEM((K, _CHUNK), jnp.int32),
            pltpu.VMEM((_CHUNK, _DEG_W), jnp.float32),
            pltpu.VMEM_SHARED((npad, _DEG_W), jnp.float32),
        ],
        compiler_params=pltpu.CompilerParams(use_tc_tiling_on_sc=False),
    )
    def deg_kernel(dst_hbm, ones_hbm, zeros_hbm, out_hbm, dst_v, ones_v, acc):
        c = lax.axis_index("c")
        s = lax.axis_index("s")
        wid = s * _NC + c
        row0 = s * rows_per_tile
        pltpu.sync_copy(dst_hbm.at[wid], dst_v)
        pltpu.sync_copy(ones_hbm, ones_v)
        pltpu.sync_copy(zeros_hbm.at[pl.ds(row0, rows_per_tile)],
                        acc.at[pl.ds(row0, rows_per_tile)])
        plsc.subcore_barrier()

        def body(j, carry):
            pltpu.sync_copy(ones_v, acc.at[dst_v.at[j]], add=True)
            return carry

        lax.fori_loop(0, K, body, 0)
        plsc.subcore_barrier()
        pltpu.sync_copy(acc.at[pl.ds(row0, rows_per_tile)],
                        out_hbm.at[c].at[pl.ds(row0, rows_per_tile)])

    return deg_kernel


def _make_agg_kernel(K, npad, rows_per_tile, d):
    @functools.partial(
        pl.kernel,
        out_type=jax.ShapeDtypeStruct((_NC, npad, d), jnp.float32),
        mesh=_sc_mesh(),
        scratch_types=[
            pltpu.VMEM((K, _CHUNK), jnp.int32),
            pltpu.VMEM((K, _CHUNK), jnp.int32),
            pltpu.VMEM((_CHUNK, d), jnp.float32),
            pltpu.VMEM_SHARED((npad, d), jnp.float32),
            pltpu.SemaphoreType.DMA,
        ],
        compiler_params=pltpu.CompilerParams(use_tc_tiling_on_sc=False),
    )
    def agg_kernel(src_hbm, dst_hbm, table_hbm, zeros_hbm, out_hbm,
                   src_v, dst_v, rows_v, acc, sem):
        c = lax.axis_index("c")
        s = lax.axis_index("s")
        wid = s * _NC + c
        row0 = s * rows_per_tile
        pltpu.sync_copy(src_hbm.at[wid], src_v)
        pltpu.sync_copy(dst_hbm.at[wid], dst_v)
        pltpu.sync_copy(zeros_hbm.at[pl.ds(row0, rows_per_tile)],
                        acc.at[pl.ds(row0, rows_per_tile)])
        plsc.subcore_barrier()

        def body(j, carry):
            pltpu.async_copy(table_hbm.at[src_v.at[j]], rows_v, sem).wait()
            pltpu.sync_copy(rows_v, acc.at[dst_v.at[j]], add=True)
            return carry

        lax.fori_loop(0, K, body, 0)
        plsc.subcore_barrier()
        pltpu.sync_copy(acc.at[pl.ds(row0, rows_per_tile)],
                        out_hbm.at[c].at[pl.ds(row0, rows_per_tile)])

    return agg_kernel


def _tc1_body(x_ref, w_ref, dparts_ref, pre_ref, r_ref):
    deg = dparts_ref[0, :, :1] + dparts_ref[1, :, :1] + 1.0
    r = lax.rsqrt(deg)
    r_ref[...] = r
    pre = jnp.dot(x_ref[...], w_ref[...], preferred_element_type=jnp.float32)
    pre_ref[...] = pre * r


def _tc2_body(a1_ref, r_ref, w_ref, pre2_ref):
    r = r_ref[...]
    h = jnp.maximum((a1_ref[0] + a1_ref[1]) * r, 0.0)
    pre2_ref[...] = jnp.dot(h, w_ref[...], preferred_element_type=jnp.float32) * r


def _tc3_body(a2_ref, r_ref, out_ref):
    out_ref[...] = (a2_ref[0] + a2_ref[1]) * r_ref[...]


def kernel(x, edge_index, W1, W2):
    n, d_feat = x.shape
    e = edge_index.shape[1]
    d_hid = W1.shape[1]
    d_out = W2.shape[1]

    d1 = 32
    d2 = 8
    npad = ((n + 127) // 128) * 128
    rows_per_tile = npad // _NS
    K = -(-e // (_NW * _CHUNK))
    epad = _NW * K * _CHUNK

    src = edge_index[0]
    dst = edge_index[1]
    src_p = jnp.concatenate([src, jnp.zeros((epad - e,), jnp.int32)])
    dst_p = jnp.concatenate([dst, jnp.full((epad - e,), n, jnp.int32)])
    src3 = src_p.reshape(_NW, K, _CHUNK)
    dst3 = dst_p.reshape(_NW, K, _CHUNK)

    x_pad = jnp.zeros((npad, d_feat), x.dtype).at[:n].set(x)
    w1p = jnp.zeros((d_feat, d1), W1.dtype).at[:, :d_hid].set(W1)
    w2p = jnp.zeros((d1, d2), W2.dtype).at[:d_hid, :d_out].set(W2)

    ones_c = jnp.ones((_CHUNK, _DEG_W), jnp.float32)
    zeros_1 = jnp.zeros((npad, _DEG_W), jnp.float32)
    zeros_d1 = jnp.zeros((npad, d1), jnp.float32)
    zeros_d2 = jnp.zeros((npad, d2), jnp.float32)

    dparts = _make_deg_kernel(K, npad, rows_per_tile)(dst3, ones_c, zeros_1)

    pre1s, r = pl.pallas_call(
        _tc1_body,
        out_shape=[
            jax.ShapeDtypeStruct((npad, d1), jnp.float32),
            jax.ShapeDtypeStruct((npad, 1), jnp.float32),
        ],
    )(x_pad, w1p, dparts)

    a1 = _make_agg_kernel(K, npad, rows_per_tile, d1)(src3, dst3, pre1s, zeros_d1)

    pre2s = pl.pallas_call(
        _tc2_body,
        out_shape=jax.ShapeDtypeStruct((npad, d2), jnp.float32),
    )(a1, r, w2p)

    a2 = _make_agg_kernel(K, npad, rows_per_tile, d2)(src3, dst3, pre2s, zeros_d2)

    out = pl.pallas_call(
        _tc3_body,
        out_shape=jax.ShapeDtypeStruct((npad, d2), jnp.float32),
    )(a2, r)

    return out[:n, :d_out]


# speedup vs baseline: 25.5492x; 1.0820x over previous
import functools

import jax
import jax.numpy as jnp
from jax import lax
from jax.experimental import pallas as pl
from jax.experimental.pallas import tpu as pltpu
from jax.experimental.pallas import tpu_sc as plsc

_NC = 2
_NS = 16
_NW = _NC * _NS
_CHUNK = 128


def _sc_mesh():
    return plsc.VectorSubcoreMesh(core_axis_name="c", subcore_axis_name="s")


_DEG_W = 8


def _make_deg_kernel(K, npad, rows_per_tile):
    @functools.partial(
        pl.kernel,
        out_type=jax.ShapeDtypeStruct((_NC, npad, _DEG_W), jnp.float32),
        mesh=_sc_mesh(),
        scratch_types=[
            pltpu.VMEM((K, _CHUNK), jnp.int32),
            pltpu.VMEM((_CHUNK, _DEG_W), jnp.float32),
            pltpu.VMEM_SHARED((npad, _DEG_W), jnp.float32),
        ],
        compiler_params=pltpu.CompilerParams(use_tc_tiling_on_sc=False),
    )
    def deg_kernel(dst_hbm, ones_hbm, zeros_hbm, out_hbm, dst_v, ones_v, acc):
        c = lax.axis_index("c")
        s = lax.axis_index("s")
        wid = s * _NC + c
        row0 = s * rows_per_tile
        pltpu.sync_copy(dst_hbm.at[wid], dst_v)
        pltpu.sync_copy(ones_hbm, ones_v)
        pltpu.sync_copy(zeros_hbm.at[pl.ds(row0, rows_per_tile)],
                        acc.at[pl.ds(row0, rows_per_tile)])
        plsc.subcore_barrier()

        def body(j, carry):
            pltpu.sync_copy(ones_v, acc.at[dst_v.at[j]], add=True)
            return carry

        lax.fori_loop(0, K, body, 0)
        plsc.subcore_barrier()
        pltpu.sync_copy(acc.at[pl.ds(row0, rows_per_tile)],
                        out_hbm.at[c].at[pl.ds(row0, rows_per_tile)])

    return deg_kernel


def _make_agg_kernel(K, npad, rows_per_tile, d):
    assert K % 2 == 0 and K >= 4

    @functools.partial(
        pl.kernel,
        out_type=jax.ShapeDtypeStruct((_NC, npad, d), jnp.float32),
        mesh=_sc_mesh(),
        scratch_types=[
            pltpu.VMEM((K, _CHUNK), jnp.int32),
            pltpu.VMEM((K, _CHUNK), jnp.int32),
            pltpu.VMEM((_CHUNK, d), jnp.float32),
            pltpu.VMEM((_CHUNK, d), jnp.float32),
            pltpu.VMEM_SHARED((npad, d), jnp.float32),
            pltpu.SemaphoreType.DMA,
            pltpu.SemaphoreType.DMA,
        ],
        compiler_params=pltpu.CompilerParams(use_tc_tiling_on_sc=False),
    )
    def agg_kernel(src_hbm, dst_hbm, table_hbm, zeros_hbm, out_hbm,
                   src_v, dst_v, rows_a, rows_b, acc, sem_a, sem_b):
        c = lax.axis_index("c")
        s = lax.axis_index("s")
        wid = s * _NC + c
        row0 = s * rows_per_tile
        pltpu.sync_copy(src_hbm.at[wid], src_v)
        pltpu.sync_copy(dst_hbm.at[wid], dst_v)
        pltpu.sync_copy(zeros_hbm.at[pl.ds(row0, rows_per_tile)],
                        acc.at[pl.ds(row0, rows_per_tile)])
        plsc.subcore_barrier()

        def gather(j, buf, sem):
            pltpu.async_copy(table_hbm.at[src_v.at[j]], buf, sem)

        def drain(buf, sem):
            pltpu.make_async_copy(table_hbm.at[src_v.at[0]], buf, sem).wait()

        gather(0, rows_a, sem_a)
        gather(1, rows_b, sem_b)

        def body(i, carry):
            j0 = 2 * i
            drain(rows_a, sem_a)
            pltpu.sync_copy(rows_a, acc.at[dst_v.at[j0]], add=True)
            gather(j0 + 2, rows_a, sem_a)
            drain(rows_b, sem_b)
            pltpu.sync_copy(rows_b, acc.at[dst_v.at[j0 + 1]], add=True)
            gather(j0 + 3, rows_b, sem_b)
            return carry

        lax.fori_loop(0, K // 2 - 1, body, 0)
        j0 = K - 2
        drain(rows_a, sem_a)
        pltpu.sync_copy(rows_a, acc.at[dst_v.at[j0]], add=True)
        drain(rows_b, sem_b)
        pltpu.sync_copy(rows_b, acc.at[dst_v.at[j0 + 1]], add=True)
        plsc.subcore_barrier()
        pltpu.sync_copy(acc.at[pl.ds(row0, rows_per_tile)],
                        out_hbm.at[c].at[pl.ds(row0, rows_per_tile)])

    return agg_kernel


def _tc1_body(x_ref, w_ref, dparts_ref, pre_ref, r_ref):
    deg = dparts_ref[0, :, :1] + dparts_ref[1, :, :1] + 1.0
    r = lax.rsqrt(deg)
    r_ref[...] = r
    pre = jnp.dot(x_ref[...], w_ref[...], preferred_element_type=jnp.float32)
    pre_ref[...] = pre * r


def _tc2_body(a1_ref, r_ref, w_ref, pre2_ref):
    r = r_ref[...]
    h = jnp.maximum((a1_ref[0] + a1_ref[1]) * r, 0.0)
    pre2_ref[...] = jnp.dot(h, w_ref[...], preferred_element_type=jnp.float32) * r


def _tc3_body(a2_ref, r_ref, out_ref):
    out_ref[...] = (a2_ref[0] + a2_ref[1]) * r_ref[...]


def kernel(x, edge_index, W1, W2):
    n, d_feat = x.shape
    e = edge_index.shape[1]
    d_hid = W1.shape[1]
    d_out = W2.shape[1]

    d1 = 32
    d2 = 8
    npad = ((n + 127) // 128) * 128
    rows_per_tile = npad // _NS
    K = -(-e // (_NW * _CHUNK))
    K += K % 2
    epad = _NW * K * _CHUNK

    src = edge_index[0]
    dst = edge_index[1]
    src_p = jnp.concatenate([src, jnp.zeros((epad - e,), jnp.int32)])
    dst_p = jnp.concatenate([dst, jnp.full((epad - e,), n, jnp.int32)])
    src3 = src_p.reshape(_NW, K, _CHUNK)
    dst3 = dst_p.reshape(_NW, K, _CHUNK)

    x_pad = jnp.zeros((npad, d_feat), x.dtype).at[:n].set(x)
    w1p = jnp.zeros((d_feat, d1), W1.dtype).at[:, :d_hid].set(W1)
    w2p = jnp.zeros((d1, d2), W2.dtype).at[:d_hid, :d_out].set(W2)

    ones_c = jnp.ones((_CHUNK, _DEG_W), jnp.float32)
    zeros_1 = jnp.zeros((npad, _DEG_W), jnp.float32)
    zeros_d1 = jnp.zeros((npad, d1), jnp.float32)
    zeros_d2 = jnp.zeros((npad, d2), jnp.float32)

    dparts = _make_deg_kernel(K, npad, rows_per_tile)(dst3, ones_c, zeros_1)

    pre1s, r = pl.pallas_call(
        _tc1_body,
        out_shape=[
            jax.ShapeDtypeStruct((npad, d1), jnp.float32),
            jax.ShapeDtypeStruct((npad, 1), jnp.float32),
        ],
    )(x_pad, w1p, dparts)

    a1 = _make_agg_kernel(K, npad, rows_per_tile, d1)(src3, dst3, pre1s, zeros_d1)

    pre2s = pl.pallas_call(
        _tc2_body,
        out_shape=jax.ShapeDtypeStruct((npad, d2), jnp.float32),
    )(a1, r, w2p)

    a2 = _make_agg_kernel(K, npad, rows_per_tile, d2)(src3, dst3, pre2s, zeros_d2)

    out = pl.pallas_call(
        _tc3_body,
        out_shape=jax.ShapeDtypeStruct((npad, d2), jnp.float32),
    )(a2, r)

    return out[:n, :d_out]


# speedup vs baseline: 27.9794x; 1.0951x over previous
import functools

import jax
import jax.numpy as jnp
from jax import lax
from jax.experimental import pallas as pl
from jax.experimental.pallas import tpu as pltpu
from jax.experimental.pallas import tpu_sc as plsc

_NC = 2
_NS = 16
_NW = _NC * _NS
_CHUNK = 128


def _sc_mesh():
    return plsc.VectorSubcoreMesh(core_axis_name="c", subcore_axis_name="s")


_DEG_W = 8


def _make_deg_kernel(K, npad, rows_per_tile):
    @functools.partial(
        pl.kernel,
        out_type=jax.ShapeDtypeStruct((_NC, npad, _DEG_W), jnp.float32),
        mesh=_sc_mesh(),
        scratch_types=[
            pltpu.VMEM((K, _CHUNK), jnp.int32),
            pltpu.VMEM((_CHUNK, _DEG_W), jnp.float32),
            pltpu.VMEM_SHARED((npad, _DEG_W), jnp.float32),
        ],
        compiler_params=pltpu.CompilerParams(use_tc_tiling_on_sc=False),
    )
    def deg_kernel(dst_hbm, ones_hbm, zeros_hbm, out_hbm, dst_v, ones_v, acc):
        c = lax.axis_index("c")
        s = lax.axis_index("s")
        wid = s * _NC + c
        row0 = s * rows_per_tile
        pltpu.sync_copy(dst_hbm.at[wid], dst_v)
        pltpu.sync_copy(ones_hbm, ones_v)
        pltpu.sync_copy(zeros_hbm.at[pl.ds(row0, rows_per_tile)],
                        acc.at[pl.ds(row0, rows_per_tile)])
        plsc.subcore_barrier()

        def body(j, carry):
            pltpu.sync_copy(ones_v, acc.at[dst_v.at[j]], add=True)
            return carry

        lax.fori_loop(0, K, body, 0)
        plsc.subcore_barrier()
        pltpu.sync_copy(acc.at[pl.ds(row0, rows_per_tile)],
                        out_hbm.at[c].at[pl.ds(row0, rows_per_tile)])

    return deg_kernel


def _make_agg_kernel(K, npad, rows_per_tile, d):
    assert K % 2 == 0 and K >= 4

    @functools.partial(
        pl.kernel,
        out_type=jax.ShapeDtypeStruct((_NC, npad, d), jnp.float32),
        mesh=_sc_mesh(),
        scratch_types=[
            pltpu.VMEM((K, _CHUNK), jnp.int32),
            pltpu.VMEM((K, _CHUNK), jnp.int32),
            pltpu.VMEM((_CHUNK, d), jnp.float32),
            pltpu.VMEM((_CHUNK, d), jnp.float32),
            pltpu.VMEM_SHARED((npad, d), jnp.float32),
            pltpu.SemaphoreType.DMA,
            pltpu.SemaphoreType.DMA,
        ],
        compiler_params=pltpu.CompilerParams(use_tc_tiling_on_sc=False),
    )
    def agg_kernel(src_hbm, dst_hbm, table_hbm, zeros_hbm, out_hbm,
                   src_v, dst_v, rows_a, rows_b, acc, sem_a, sem_b):
        c = lax.axis_index("c")
        s = lax.axis_index("s")
        wid = s * _NC + c
        row0 = s * rows_per_tile
        pltpu.sync_copy(src_hbm.at[wid], src_v)
        pltpu.sync_copy(dst_hbm.at[wid], dst_v)
        pltpu.sync_copy(zeros_hbm.at[pl.ds(row0, rows_per_tile)],
                        acc.at[pl.ds(row0, rows_per_tile)])
        plsc.subcore_barrier()

        def gather(j, buf, sem):
            pltpu.async_copy(table_hbm.at[src_v.at[j]], buf, sem)

        def drain(buf, sem):
            pltpu.make_async_copy(table_hbm.at[src_v.at[0]], buf, sem).wait()

        gather(0, rows_a, sem_a)
        gather(1, rows_b, sem_b)

        def body(i, carry):
            j0 = 2 * i
            drain(rows_a, sem_a)
            pltpu.sync_copy(rows_a, acc.at[dst_v.at[j0]], add=True)
            gather(j0 + 2, rows_a, sem_a)
            drain(rows_b, sem_b)
            pltpu.sync_copy(rows_b, acc.at[dst_v.at[j0 + 1]], add=True)
            gather(j0 + 3, rows_b, sem_b)
            return carry

        lax.fori_loop(0, K // 2 - 1, body, 0)
        j0 = K - 2
        drain(rows_a, sem_a)
        pltpu.sync_copy(rows_a, acc.at[dst_v.at[j0]], add=True)
        drain(rows_b, sem_b)
        pltpu.sync_copy(rows_b, acc.at[dst_v.at[j0 + 1]], add=True)
        plsc.subcore_barrier()
        pltpu.sync_copy(acc.at[pl.ds(row0, rows_per_tile)],
                        out_hbm.at[c].at[pl.ds(row0, rows_per_tile)])

    return agg_kernel


def _tc1_body(x_ref, w_ref, dparts_ref, pre_ref, r_ref):
    deg = dparts_ref[0, :, :1] + dparts_ref[1, :, :1] + 1.0
    r = lax.rsqrt(deg)
    r_ref[...] = r
    pre = jnp.dot(x_ref[...], w_ref[...], preferred_element_type=jnp.float32)
    pre_ref[...] = pre * r


def _tc2_body(a1_ref, r_ref, w_ref, pre2_ref):
    r = r_ref[...]
    h = jnp.maximum((a1_ref[0] + a1_ref[1]) * r, 0.0)
    pre2_ref[...] = jnp.dot(h, w_ref[...], preferred_element_type=jnp.float32) * r


def _tc3_body(a2_ref, r_ref, out_ref):
    out_ref[...] = (a2_ref[0] + a2_ref[1]) * r_ref[...]


def kernel(x, edge_index, W1, W2):
    n, d_feat = x.shape
    e = edge_index.shape[1]
    d_hid = W1.shape[1]
    d_out = W2.shape[1]

    d1 = 24
    d2 = 8
    npad = ((n + 127) // 128) * 128
    rows_per_tile = npad // _NS
    K = -(-e // (_NW * _CHUNK))
    K += K % 2
    epad = _NW * K * _CHUNK

    src = edge_index[0]
    dst = edge_index[1]
    src_p = jnp.concatenate([src, jnp.zeros((epad - e,), jnp.int32)])
    dst_p = jnp.concatenate([dst, jnp.full((epad - e,), n, jnp.int32)])
    src3 = src_p.reshape(_NW, K, _CHUNK)
    dst3 = dst_p.reshape(_NW, K, _CHUNK)

    x_pad = jnp.zeros((npad, d_feat), x.dtype).at[:n].set(x)
    w1p = jnp.zeros((d_feat, d1), W1.dtype).at[:, :d_hid].set(W1)
    w2p = jnp.zeros((d1, d2), W2.dtype).at[:d_hid, :d_out].set(W2)

    ones_c = jnp.ones((_CHUNK, _DEG_W), jnp.float32)
    zeros_1 = jnp.zeros((npad, _DEG_W), jnp.float32)
    zeros_d1 = jnp.zeros((npad, d1), jnp.float32)
    zeros_d2 = jnp.zeros((npad, d2), jnp.float32)

    dparts = _make_deg_kernel(K, npad, rows_per_tile)(dst3, ones_c, zeros_1)

    pre1s, r = pl.pallas_call(
        _tc1_body,
        out_shape=[
            jax.ShapeDtypeStruct((npad, d1), jnp.float32),
            jax.ShapeDtypeStruct((npad, 1), jnp.float32),
        ],
    )(x_pad, w1p, dparts)

    a1 = _make_agg_kernel(K, npad, rows_per_tile, d1)(src3, dst3, pre1s, zeros_d1)

    pre2s = pl.pallas_call(
        _tc2_body,
        out_shape=jax.ShapeDtypeStruct((npad, d2), jnp.float32),
    )(a1, r, w2p)

    a2 = _make_agg_kernel(K, npad, rows_per_tile, d2)(src3, dst3, pre2s, zeros_d2)

    out = pl.pallas_call(
        _tc3_body,
        out_shape=jax.ShapeDtypeStruct((npad, d2), jnp.float32),
    )(a2, r)

    return out[:n, :d_out]
